# Initial kernel scaffold; baseline (speedup 1.0000x reference)
import functools

import jax
import jax.numpy as jnp
from jax import lax
from jax.experimental import pallas as pl
from jax.experimental.pallas import tpu as pltpu
from jax.experimental.pallas import tpu_sc as plsc

N = 10000
D = 128
K = 128
NC = 2
NS = 16
W = NC * NS
ROWS_PER_TILE = N // NS


def _spmm_sc(table, cols, rows, vals_exp, zeros):
    e_pad = cols.shape[0]
    ch = e_pad // (W * K)
    mesh = plsc.VectorSubcoreMesh(core_axis_name="c", subcore_axis_name="s")

    @functools.partial(
        pl.kernel,
        out_type=jax.ShapeDtypeStruct((NC * N, D), jnp.float32),
        mesh=mesh,
        scratch_types=[
            pltpu.VMEM((K,), jnp.int32),
            pltpu.VMEM((K,), jnp.int32),
            pltpu.VMEM((K, 16), jnp.float32),
            pltpu.VMEM((K, D), jnp.float32),
            pltpu.VMEM_SHARED((N, D), jnp.float32),
            pltpu.SemaphoreType.DMA,
        ],
    )
    def spmm(table_h, cols_h, rows_h, vals_h, zeros_h, out_h,
             colbuf, rowbuf, valbuf, gbuf, acc, sem):
        cid = lax.axis_index("c")
        sid = lax.axis_index("s")
        wid = sid * NC + cid
        r0 = sid * ROWS_PER_TILE
        pltpu.sync_copy(zeros_h.at[pl.ds(r0, ROWS_PER_TILE)],
                        acc.at[pl.ds(r0, ROWS_PER_TILE)])
        plsc.subcore_barrier()

        def chunk(i, carry):
            base = (wid * ch + i) * K
            pltpu.sync_copy(cols_h.at[pl.ds(base, K)], colbuf)
            pltpu.sync_copy(rows_h.at[pl.ds(base, K)], rowbuf)
            pltpu.sync_copy(vals_h.at[pl.ds(base, K)], valbuf)
            pltpu.async_copy(table_h.at[colbuf], gbuf, sem).wait()

            def edge(k, c2):
                splat = valbuf[k, :]
                for u in range(D // 16):
                    gbuf[k, pl.ds(u * 16, 16)] = (
                        gbuf[k, pl.ds(u * 16, 16)] * splat)
                return c2

            lax.fori_loop(0, K, edge, 0)
            pltpu.sync_copy(gbuf, acc.at[rowbuf], add=True)
            return carry

        lax.fori_loop(0, ch, chunk, 0)
        plsc.subcore_barrier()
        pltpu.sync_copy(acc.at[pl.ds(r0, ROWS_PER_TILE)],
                        out_h.at[pl.ds(cid * N + r0, ROWS_PER_TILE)])

    return spmm(table, cols, rows, vals_exp, zeros)


_BN = 2000


def _tanh_combine(p):
    def body(p0_ref, p1_ref, o_ref):
        o_ref[...] = jnp.tanh(p0_ref[...] + p1_ref[...])

    return pl.pallas_call(
        body,
        grid=(N // _BN,),
        in_specs=[pl.BlockSpec((_BN, D), lambda i: (i, 0)),
                  pl.BlockSpec((_BN, D), lambda i: (i, 0))],
        out_specs=pl.BlockSpec((_BN, D), lambda i: (i, 0)),
        out_shape=jax.ShapeDtypeStruct((N, D), jnp.float32),
    )(p[:N], p[N:])


def _final_sum(x, t1, p):
    def body(x_ref, t1_ref, p0_ref, p1_ref, o_ref):
        o_ref[...] = (x_ref[...] + t1_ref[...]
                      + jnp.tanh(p0_ref[...] + p1_ref[...]))

    return pl.pallas_call(
        body,
        grid=(N // _BN,),
        in_specs=[pl.BlockSpec((_BN, D), lambda i: (i, 0))] * 4,
        out_specs=pl.BlockSpec((_BN, D), lambda i: (i, 0)),
        out_shape=jax.ShapeDtypeStruct((N, D), jnp.float32),
    )(x, t1, p[:N], p[N:])


def kernel(inputs_weight, support_indices, support_values):
    x = inputs_weight[1:]
    rows = support_indices[0]
    cols = support_indices[1]
    vals = support_values
    e = vals.shape[0]
    ch = -(-e // (W * K))
    e_pad = W * K * ch
    pad = e_pad - e
    cols_p = jnp.pad(cols, (0, pad))
    rows_p = jnp.pad(rows, (0, pad))
    vals_p = jnp.pad(vals, (0, pad))
    vals_exp = jnp.broadcast_to(vals_p[:, None], (e_pad, 16))
    zeros = jnp.zeros((N, D), jnp.float32)

    p1 = _spmm_sc(x, cols_p, rows_p, vals_exp, zeros)
    t1 = _tanh_combine(p1)
    p2 = _spmm_sc(t1, cols_p, rows_p, vals_exp, zeros)
    out = _final_sum(x, t1, p2)
    return jnp.concatenate([inputs_weight[0:1], out], axis=0)


# speedup vs baseline: 3.0260x; 3.0260x over previous
import functools

import jax
import jax.numpy as jnp
from jax import lax
from jax.experimental import pallas as pl
from jax.experimental.pallas import tpu as pltpu
from jax.experimental.pallas import tpu_sc as plsc

N = 10000
NP = 10240
D = 128
K = 128
NC = 2
NS = 16
W = NC * NS
ROWS_PER_TILE = NP // NS


def _spmm_sc(table, cols, rows, vals_exp, zeros):
    e_pad = cols.shape[0]
    ch = e_pad // (W * K)
    mesh = plsc.VectorSubcoreMesh(core_axis_name="c", subcore_axis_name="s")

    @functools.partial(
        pl.kernel,
        out_type=jax.ShapeDtypeStruct((NC * NP, D), jnp.float32),
        mesh=mesh,
        scratch_types=[
            pltpu.VMEM((K,), jnp.int32),
            pltpu.VMEM((K,), jnp.int32),
            pltpu.VMEM((K, 16), jnp.float32),
            pltpu.VMEM((K, D), jnp.float32),
            pltpu.VMEM_SHARED((NP, D), jnp.float32),
            pltpu.SemaphoreType.DMA,
        ],
    )
    def spmm(table_h, cols_h, rows_h, vals_h, zeros_h, out_h,
             colbuf, rowbuf, valbuf, gbuf, acc, sem):
        cid = lax.axis_index("c")
        sid = lax.axis_index("s")
        wid = sid * NC + cid
        r0 = sid * ROWS_PER_TILE
        pltpu.sync_copy(zeros_h.at[pl.ds(r0, ROWS_PER_TILE)],
                        acc.at[pl.ds(r0, ROWS_PER_TILE)])
        plsc.subcore_barrier()

        def chunk(i, carry):
            base = (wid * ch + i) * K
            pltpu.sync_copy(cols_h.at[pl.ds(base, K)], colbuf)
            pltpu.sync_copy(rows_h.at[pl.ds(base, K)], rowbuf)
            pltpu.sync_copy(vals_h.at[pl.ds(base, K)], valbuf)
            pltpu.async_copy(table_h.at[colbuf], gbuf, sem).wait()

            def edge(k, c2):
                splat = valbuf[k, :]
                for u in range(D // 16):
                    gbuf[k, pl.ds(u * 16, 16)] = (
                        gbuf[k, pl.ds(u * 16, 16)] * splat)
                return c2

            lax.fori_loop(0, K, edge, 0)
            pltpu.sync_copy(gbuf, acc.at[rowbuf], add=True)
            return carry

        lax.fori_loop(0, ch, chunk, 0)
        plsc.subcore_barrier()
        pltpu.sync_copy(acc.at[pl.ds(r0, ROWS_PER_TILE)],
                        out_h.at[pl.ds(cid * NP + r0, ROWS_PER_TILE)])

    return spmm(table, cols, rows, vals_exp, zeros)


_BN = 2000


def _tanh_combine(p):
    def body(p0_ref, p1_ref, o_ref):
        o_ref[...] = jnp.tanh(p0_ref[...] + p1_ref[...])

    return pl.pallas_call(
        body,
        grid=(N // _BN,),
        in_specs=[pl.BlockSpec((_BN, D), lambda i: (i, 0)),
                  pl.BlockSpec((_BN, D), lambda i: (i, 0))],
        out_specs=pl.BlockSpec((_BN, D), lambda i: (i, 0)),
        out_shape=jax.ShapeDtypeStruct((N, D), jnp.float32),
    )(p[:N], p[NP:NP + N])


def _final_sum(x, t1, p):
    def body(x_ref, t1_ref, p0_ref, p1_ref, o_ref):
        o_ref[...] = (x_ref[...] + t1_ref[...]
                      + jnp.tanh(p0_ref[...] + p1_ref[...]))

    return pl.pallas_call(
        body,
        grid=(N // _BN,),
        in_specs=[pl.BlockSpec((_BN, D), lambda i: (i, 0))] * 4,
        out_specs=pl.BlockSpec((_BN, D), lambda i: (i, 0)),
        out_shape=jax.ShapeDtypeStruct((N, D), jnp.float32),
    )(x, t1, p[:N], p[NP:NP + N])


def kernel(inputs_weight, support_indices, support_values):
    x = inputs_weight[1:]
    rows = support_indices[0]
    cols = support_indices[1]
    vals = support_values
    e = vals.shape[0]
    ch = -(-e // (W * K))
    e_pad = W * K * ch
    pad = e_pad - e
    cols_p = jnp.pad(cols, (0, pad))
    rows_p = jnp.pad(rows, (0, pad))
    vals_p = jnp.pad(vals, (0, pad))
    vals_exp = jnp.broadcast_to(vals_p[:, None], (e_pad, 16))
    zeros = jnp.zeros((NP, D), jnp.float32)

    p1 = _spmm_sc(x, cols_p, rows_p, vals_exp, zeros)
    t1 = _tanh_combine(p1)
    p2 = _spmm_sc(t1, cols_p, rows_p, vals_exp, zeros)
    out = _final_sum(x, t1, p2)
    return jnp.concatenate([inputs_weight[0:1], out], axis=0)
